# baseline (speedup 1.0000x reference)
import jax
import jax.numpy as jnp
from jax import lax
from jax.experimental import pallas as pl
from jax.experimental.pallas import tpu as pltpu
from jax.experimental.pallas import tpu_sc as plsc

B = 16384
L = 20
D = 32
NW = 32
NI = 1000000

TCW = 512
NB = (NI + TCW - 1) // TCW

BPW = B // NW
G = 128
CB = 64
NCH = BPW // CB
CROWS = CB * L
CG = CROWS // G
IROWS_W = BPW * L // G
IROWS_I = BPW // G


def _tr_block(vt_ref, out_ref):
  t = vt_ref[...].T.reshape(TCW // 4, 4, D)
  for k in range(4):
    out_ref[:, k * D:(k + 1) * D] = t[:, k, :]


def _transpose_tc(vt):
  return pl.pallas_call(
      _tr_block,
      grid=(NB,),
      in_specs=[pl.BlockSpec((32, TCW), lambda b: (0, b))],
      out_specs=pl.BlockSpec((TCW // 4, 128), lambda b: (b, 0)),
      out_shape=jax.ShapeDtypeStruct((NI // 4, 128), jnp.float32),
  )(vt)


def _gather_body(i_hbm, lb_hbm, vil, vli, out_hbm,
                 i_v, lb_v, ei_v, el_a, el_b, tbuf, out_v,
                 sem_ei, sem_a, sem_b):
  w = lax.axis_index("s") * 2 + lax.axis_index("c")
  base = w * BPW

  pltpu.sync_copy(i_hbm.at[pl.ds(w * IROWS_I, IROWS_I)], i_v)
  pltpu.sync_copy(lb_hbm.at[pl.ds(w * IROWS_W, IROWS_W)], lb_v)

  ei_copies = [
      pltpu.async_copy(vil.at[i_v.at[j]], ei_v.at[pl.ds(j * G, G)], sem_ei)
      for j in range(IROWS_I)
  ]

  el_bufs = (el_a, el_b)
  sems = (sem_a, sem_b)

  def start_chunk(c):
    p = c % 2
    return [
        pltpu.async_copy(vli.at[lb_v.at[c * CG + j]],
                         el_bufs[p].at[pl.ds(j * G, G)], sems[p])
        for j in range(CG)
    ]

  pending = {0: start_chunk(0)}
  for cp in ei_copies:
    cp.wait()

  for c in range(NCH):
    if c + 1 < NCH:
      pending[c + 1] = start_chunk(c + 1)
    for cp in pending.pop(c):
      cp.wait()
    el = el_bufs[c % 2]

    def bbody(b, carry, el=el, c=c):
      r0 = b * L
      s0 = el[r0, pl.ds(0, 16)]
      s1 = el[r0, pl.ds(16, 16)]
      for l in range(1, L):
        s0 = s0 + el[r0 + l, pl.ds(0, 16)]
        s1 = s1 + el[r0 + l, pl.ds(16, 16)]
      cb = c * CB + b
      t = ei_v[cb, pl.ds(0, 16)] * s0 + ei_v[cb, pl.ds(16, 16)] * s1
      tbuf[b, :] = t
      return carry

    lax.fori_loop(0, CB, bbody, 0)

    lane = jnp.arange(16, dtype=jnp.int32)
    for bg in range(CB // 16):
      rows = lane + (bg * 16)
      acc = plsc.load_gather(tbuf, [rows, jnp.full((16,), 0, jnp.int32)])
      for k in range(1, 16):
        acc = acc + plsc.load_gather(tbuf, [rows, jnp.full((16,), k, jnp.int32)])
      out_v[pl.ds(c * CB + bg * 16, 16)] = acc * jnp.float32(1.0 / L)

  pltpu.sync_copy(out_v, out_hbm.at[pl.ds(base, BPW)])


def _fpmc(i2, lb2, vil, vli):
  mesh = plsc.VectorSubcoreMesh(core_axis_name="c", subcore_axis_name="s")
  return pl.kernel(
      _gather_body,
      out_type=jax.ShapeDtypeStruct((B,), jnp.float32),
      mesh=mesh,
      compiler_params=pltpu.CompilerParams(
          needs_layout_passes=False, use_tc_tiling_on_sc=False),
      scratch_types=[
          pltpu.VMEM((IROWS_I, G), jnp.int32),
          pltpu.VMEM((IROWS_W, G), jnp.int32),
          pltpu.VMEM((BPW, D), jnp.float32),
          pltpu.VMEM((CROWS, D), jnp.float32),
          pltpu.VMEM((CROWS, D), jnp.float32),
          pltpu.VMEM((CB, 16), jnp.float32),
          pltpu.VMEM((BPW,), jnp.float32),
          pltpu.SemaphoreType.DMA,
          pltpu.SemaphoreType.DMA,
          pltpu.SemaphoreType.DMA,
      ],
  )(i2, lb2, vil, vli)


def kernel(u, i, last_basket, V_IL, V_LI):
  del u
  t_il = _transpose_tc(V_IL.T)
  t_li = _transpose_tc(V_LI.T)
  i2 = i.astype(jnp.int32).reshape(B // G, G)
  lb2 = last_basket.astype(jnp.int32).reshape(B * L // G, G)
  return _fpmc(i2, lb2, t_il.reshape(NI, D), t_li.reshape(NI, D))


# speedup vs baseline: 5.4988x; 5.4988x over previous
import jax
import jax.numpy as jnp
from jax import lax
from jax.experimental import pallas as pl
from jax.experimental.pallas import tpu as pltpu
from jax.experimental.pallas import tpu_sc as plsc

B = 16384
L = 20
D = 32
NW = 32
NI = 1000000

NGRP = NI // 128
TAIL = NI - NGRP * 128
KPW = NGRP // NW
NPAIR = KPW // 2
LEFT0 = KPW * NW

BPW = B // NW
G = 128
CB = 64
NCH = BPW // CB
CROWS = CB * L
CG = CROWS // G
IROWS_W = BPW * L // G


def _shuffle_group(vin, skew, vout, width):
  lane = jnp.arange(16, dtype=jnp.int32)

  def rot(d, carry):
    dv = jnp.broadcast_to(d, (16,)).astype(jnp.int32)
    for ib in range(width // 16):
      col = lax.bitwise_and(lane + (ib * 16) + dv * 5, 127)
      plsc.store_scatter(skew, [dv, col], vin[d, pl.ds(ib * 16, 16)])
    return carry

  lax.fori_loop(0, 32, rot, 0)

  def emit(s, carry):
    for di in range(4):
      i = s * 4 + di
      iv = jnp.broadcast_to(i, (16,)).astype(jnp.int32)
      lo = plsc.load_gather(skew, [lane, lax.bitwise_and(iv + lane * 5, 127)])
      hi = plsc.load_gather(
          skew, [lane + 16, lax.bitwise_and(iv + (lane + 16) * 5, 127)])
      r = lax.shift_right_logical(i, 2)
      c0 = lax.mul(lax.bitwise_and(i, 3), 32)
      vout[r, pl.ds(c0, 16)] = lo
      vout[r, pl.ds(c0 + 16, 16)] = hi
    return carry

  lax.fori_loop(0, width // 4, emit, 0)


def _tr_body(vt, tail_rows, t,
             vin_a, vin_b, skew, vout_a, vout_b, vin_t, vout_t,
             sem_ia, sem_ib, sem_oa, sem_ob):
  w = lax.axis_index("s") * 2 + lax.axis_index("c")

  pltpu.async_copy(vt.at[:, pl.ds(w * 128, 128)], vin_a, sem_ia)

  def pair(p, carry):
    g0 = w + p * 64
    g1 = g0 + 32
    g2 = jnp.minimum(g0 + 64, NGRP - 1)
    pltpu.async_copy(vt.at[:, pl.ds(g1 * 128, 128)], vin_b, sem_ib)

    @pl.when(p > 0)
    def _():
      pltpu.make_async_copy(vout_a, t.at[pl.ds(0, 32)], sem_oa).wait()
    pltpu.make_async_copy(vt.at[:, pl.ds(0, 128)], vin_a, sem_ia).wait()
    _shuffle_group(vin_a, skew, vout_a, 128)
    pltpu.async_copy(vout_a, t.at[pl.ds(g0 * 32, 32)], sem_oa)
    pltpu.async_copy(vt.at[:, pl.ds(g2 * 128, 128)], vin_a, sem_ia)

    @pl.when(p > 0)
    def _():
      pltpu.make_async_copy(vout_b, t.at[pl.ds(0, 32)], sem_ob).wait()
    pltpu.make_async_copy(vt.at[:, pl.ds(0, 128)], vin_b, sem_ib).wait()
    _shuffle_group(vin_b, skew, vout_b, 128)
    pltpu.async_copy(vout_b, t.at[pl.ds(g1 * 32, 32)], sem_ob)
    return carry

  lax.fori_loop(0, NPAIR, pair, 0)

  pltpu.make_async_copy(vt.at[:, pl.ds(0, 128)], vin_a, sem_ia).wait()
  pltpu.make_async_copy(vout_a, t.at[pl.ds(0, 32)], sem_oa).wait()
  pltpu.make_async_copy(vout_b, t.at[pl.ds(0, 32)], sem_ob).wait()

  @pl.when(w < NGRP - LEFT0)
  def _():
    g = LEFT0 + w
    pltpu.sync_copy(vt.at[:, pl.ds(g * 128, 128)], vin_a)
    _shuffle_group(vin_a, skew, vout_a, 128)
    pltpu.sync_copy(vout_a, t.at[pl.ds(g * 32, 32)])

  @pl.when(w == 4)
  def _():
    pltpu.sync_copy(tail_rows, vin_t)

    def pack(s, carry):
      for di in range(4):
        i = s * 4 + di
        r = lax.shift_right_logical(i, 2)
        c0 = lax.mul(lax.bitwise_and(i, 3), 32)
        vout_t[r, pl.ds(c0, 16)] = vin_t[i, pl.ds(0, 16)]
        vout_t[r, pl.ds(c0 + 16, 16)] = vin_t[i, pl.ds(16, 16)]
      return carry

    lax.fori_loop(0, TAIL // 4, pack, 0)
    pltpu.sync_copy(vout_t, t.at[pl.ds(NGRP * 32, TAIL // 4)])


def _transpose_li(vt_li, tail_rows):
  mesh = plsc.VectorSubcoreMesh(core_axis_name="c", subcore_axis_name="s")
  return pl.kernel(
      _tr_body,
      out_type=jax.ShapeDtypeStruct((NI // 4, 128), jnp.float32),
      mesh=mesh,
      compiler_params=pltpu.CompilerParams(
          needs_layout_passes=False, use_tc_tiling_on_sc=True),
      scratch_types=[
          pltpu.VMEM((32, 128), jnp.float32),
          pltpu.VMEM((32, 128), jnp.float32),
          pltpu.VMEM((32, 128), jnp.float32),
          pltpu.VMEM((32, 128), jnp.float32),
          pltpu.VMEM((32, 128), jnp.float32),
          pltpu.VMEM((TAIL, 128), jnp.float32),
          pltpu.VMEM((TAIL // 4, 128), jnp.float32),
          pltpu.SemaphoreType.DMA,
          pltpu.SemaphoreType.DMA,
          pltpu.SemaphoreType.DMA,
          pltpu.SemaphoreType.DMA,
      ],
  )(vt_li, tail_rows)


def _gather_body(ei_hbm, lb_hbm, vli, out_hbm,
                 lb_v, ei_v, el_a, el_b, tbuf, out_v,
                 sem_ei, sem_a, sem_b):
  w = lax.axis_index("s") * 2 + lax.axis_index("c")
  base = w * BPW

  pltpu.sync_copy(lb_hbm.at[pl.ds(w * IROWS_W, IROWS_W)], lb_v)
  ei_cp = pltpu.async_copy(ei_hbm.at[pl.ds(base, BPW)], ei_v, sem_ei)

  el_bufs = (el_a, el_b)
  sems = (sem_a, sem_b)

  def start_chunk(c):
    p = c % 2
    return [
        pltpu.async_copy(vli.at[lb_v.at[c * CG + j]],
                         el_bufs[p].at[pl.ds(j * G, G)], sems[p])
        for j in range(CG)
    ]

  pending = {0: start_chunk(0)}
  ei_cp.wait()

  for c in range(NCH):
    if c + 1 < NCH:
      pending[c + 1] = start_chunk(c + 1)
    for cp in pending.pop(c):
      cp.wait()
    el = el_bufs[c % 2]

    def bbody(b, carry, el=el, c=c):
      r0 = b * L
      s0 = el[r0, pl.ds(0, 16)]
      s1 = el[r0, pl.ds(16, 16)]
      for l in range(1, L):
        s0 = s0 + el[r0 + l, pl.ds(0, 16)]
        s1 = s1 + el[r0 + l, pl.ds(16, 16)]
      cb = c * CB + b
      t = ei_v[cb, pl.ds(0, 16)] * s0 + ei_v[cb, pl.ds(16, 16)] * s1
      tbuf[b, :] = t
      return carry

    lax.fori_loop(0, CB, bbody, 0)

    lane = jnp.arange(16, dtype=jnp.int32)
    for bg in range(CB // 16):
      rows = lane + (bg * 16)
      acc = plsc.load_gather(tbuf, [rows, jnp.full((16,), 0, jnp.int32)])
      for k in range(1, 16):
        acc = acc + plsc.load_gather(tbuf, [rows, jnp.full((16,), k, jnp.int32)])
      out_v[pl.ds(c * CB + bg * 16, 16)] = acc * jnp.float32(1.0 / L)

  pltpu.sync_copy(out_v, out_hbm.at[pl.ds(base, BPW)])


def _fpmc(ei, lb2, vli):
  mesh = plsc.VectorSubcoreMesh(core_axis_name="c", subcore_axis_name="s")
  return pl.kernel(
      _gather_body,
      out_type=jax.ShapeDtypeStruct((B,), jnp.float32),
      mesh=mesh,
      compiler_params=pltpu.CompilerParams(
          needs_layout_passes=False, use_tc_tiling_on_sc=False),
      scratch_types=[
          pltpu.VMEM((IROWS_W, G), jnp.int32),
          pltpu.VMEM((BPW, D), jnp.float32),
          pltpu.VMEM((CROWS, D), jnp.float32),
          pltpu.VMEM((CROWS, D), jnp.float32),
          pltpu.VMEM((CB, 16), jnp.float32),
          pltpu.VMEM((BPW,), jnp.float32),
          pltpu.SemaphoreType.DMA,
          pltpu.SemaphoreType.DMA,
          pltpu.SemaphoreType.DMA,
      ],
  )(ei, lb2, vli)


def kernel(u, i, last_basket, V_IL, V_LI):
  del u
  tail_rows = jnp.zeros((TAIL, 128), jnp.float32).at[:, :D].set(
      V_LI[NGRP * 128:])
  t_li = _transpose_li(V_LI.T, tail_rows)
  e_i = jnp.take(V_IL, i, axis=0)
  lb2 = last_basket.astype(jnp.int32).reshape(B * L // G, G)
  return _fpmc(e_i, lb2, t_li.reshape(NI, D))
